# Initial kernel scaffold; baseline (speedup 1.0000x reference)
import jax
import jax.numpy as jnp
from jax.experimental import pallas as pl


def kernel(x, edge_index, W1, b1, W2, b2):
    raise NotImplementedError("write your pallas kernel here")


# speedup vs baseline: 20.0953x; 20.0953x over previous
import functools

import jax
import jax.numpy as jnp
from jax import lax
from jax.experimental import pallas as pl
from jax.experimental.pallas import tpu as pltpu
from jax.experimental.pallas import tpu_sc as plsc

N = 10000
E = 320000
D = 128

NC = 2
NS = 16
NW = NC * NS
EW = E // NW
C = 125
NCH = EW // C
TOTCH = E // C
RA = 624
TAIL = N - NS * RA

_MESH = plsc.VectorSubcoreMesh(
    core_axis_name="c", subcore_axis_name="s", num_cores=NC, num_subcores=NS)


def _sc_hist_body(dst_hbm, ones_hbm, z_hbm, out0, out1,
                  idx_d, ones_v, acc):
    cid = lax.axis_index("c")
    sid = lax.axis_index("s")
    wid = sid * NC + cid
    rsl = pl.ds(sid * RA, RA)
    tsl = pl.ds(NS * RA, TAIL)
    pltpu.sync_copy(z_hbm.at[rsl], acc.at[rsl])

    @pl.when(sid == NS - 1)
    def _():
        pltpu.sync_copy(z_hbm.at[tsl], acc.at[tsl])

    pltpu.sync_copy(dst_hbm.at[pl.ds(wid * NCH, NCH)], idx_d)
    pltpu.sync_copy(ones_hbm, ones_v)
    plsc.subcore_barrier()

    def body(j, carry):
        pltpu.sync_copy(ones_v, acc.at[idx_d.at[j]], add=True)
        return carry

    lax.fori_loop(0, NCH, body, 0)
    plsc.subcore_barrier()

    @pl.when(cid == 0)
    def _():
        pltpu.sync_copy(acc.at[rsl], out0.at[rsl])

        @pl.when(sid == NS - 1)
        def _():
            pltpu.sync_copy(acc.at[tsl], out0.at[tsl])

    @pl.when(cid == 1)
    def _():
        pltpu.sync_copy(acc.at[rsl], out1.at[rsl])

        @pl.when(sid == NS - 1)
        def _():
            pltpu.sync_copy(acc.at[tsl], out1.at[tsl])


_sc_hist = pl.kernel(
    _sc_hist_body,
    out_type=[jax.ShapeDtypeStruct((N, D), jnp.float32),
              jax.ShapeDtypeStruct((N, D), jnp.float32)],
    mesh=_MESH,
    scratch_types=[
        pltpu.VMEM((NCH, C), jnp.int32),
        pltpu.VMEM((C, D), jnp.float32),
        pltpu.VMEM_SHARED((N, D), jnp.float32),
    ],
)


def _sc_agg_body(src_hbm, dst_hbm, g_hbm, z_hbm, out0, out1,
                 idx_s, idx_d, rows, acc):
    cid = lax.axis_index("c")
    sid = lax.axis_index("s")
    wid = sid * NC + cid
    rsl = pl.ds(sid * RA, RA)
    tsl = pl.ds(NS * RA, TAIL)
    @pl.when(cid == 0)
    def _():
        pltpu.sync_copy(g_hbm.at[rsl], acc.at[rsl])

        @pl.when(sid == NS - 1)
        def _():
            pltpu.sync_copy(g_hbm.at[tsl], acc.at[tsl])

    @pl.when(cid == 1)
    def _():
        pltpu.sync_copy(z_hbm.at[rsl], acc.at[rsl])

        @pl.when(sid == NS - 1)
        def _():
            pltpu.sync_copy(z_hbm.at[tsl], acc.at[tsl])

    pltpu.sync_copy(src_hbm.at[pl.ds(wid * NCH, NCH)], idx_s)
    pltpu.sync_copy(dst_hbm.at[pl.ds(wid * NCH, NCH)], idx_d)
    plsc.subcore_barrier()

    def body(j, carry):
        pltpu.sync_copy(g_hbm.at[idx_s.at[j]], rows)
        pltpu.sync_copy(rows, acc.at[idx_d.at[j]], add=True)
        return carry

    lax.fori_loop(0, NCH, body, 0)
    plsc.subcore_barrier()

    @pl.when(cid == 0)
    def _():
        pltpu.sync_copy(acc.at[rsl], out0.at[rsl])

        @pl.when(sid == NS - 1)
        def _():
            pltpu.sync_copy(acc.at[tsl], out0.at[tsl])

    @pl.when(cid == 1)
    def _():
        pltpu.sync_copy(acc.at[rsl], out1.at[rsl])

        @pl.when(sid == NS - 1)
        def _():
            pltpu.sync_copy(acc.at[tsl], out1.at[tsl])


_sc_agg = pl.kernel(
    _sc_agg_body,
    out_type=[jax.ShapeDtypeStruct((N, D), jnp.float32),
              jax.ShapeDtypeStruct((N, D), jnp.float32)],
    mesh=_MESH,
    scratch_types=[
        pltpu.VMEM((NCH, C), jnp.int32),
        pltpu.VMEM((NCH, C), jnp.int32),
        pltpu.VMEM((C, D), jnp.float32),
        pltpu.VMEM_SHARED((N, D), jnp.float32),
    ],
)


_R = 1000


def _tc_pre_body(h0_ref, h1_ref, x_ref, w_ref, g_ref, dv_ref):
    deg = h0_ref[:, 0:1] + h1_ref[:, 0:1] + 1.0
    dinv = lax.rsqrt(jnp.maximum(deg, 1.0))
    h = jnp.dot(x_ref[...], w_ref[...], preferred_element_type=jnp.float32)
    g_ref[...] = h * dinv
    dv_ref[...] = jnp.broadcast_to(dinv, (_R, 16))


def _tc_pre(h0, h1, x, W1):
    return pl.pallas_call(
        _tc_pre_body,
        grid=(N // _R,),
        in_specs=[
            pl.BlockSpec((_R, D), lambda i: (i, 0)),
            pl.BlockSpec((_R, D), lambda i: (i, 0)),
            pl.BlockSpec((_R, D), lambda i: (i, 0)),
            pl.BlockSpec((D, D), lambda i: (0, 0)),
        ],
        out_specs=[pl.BlockSpec((_R, D), lambda i: (i, 0)),
                   pl.BlockSpec((_R, 16), lambda i: (i, 0))],
        out_shape=[jax.ShapeDtypeStruct((N, D), jnp.float32),
                   jax.ShapeDtypeStruct((N, 16), jnp.float32)],
    )(h0, h1, x, W1)


def _tc_mid_body(p0_ref, p1_ref, dv_ref, b_ref, w_ref, g_ref):
    dinv = dv_ref[:, 0:1]
    z = jnp.maximum((p0_ref[...] + p1_ref[...]) * dinv + b_ref[...], 0.0)
    g_ref[...] = jnp.dot(
        z, w_ref[...], preferred_element_type=jnp.float32) * dinv


def _tc_mid(p0, p1, dv, b1, W2):
    return pl.pallas_call(
        _tc_mid_body,
        grid=(N // _R,),
        in_specs=[
            pl.BlockSpec((_R, D), lambda i: (i, 0)),
            pl.BlockSpec((_R, D), lambda i: (i, 0)),
            pl.BlockSpec((_R, 16), lambda i: (i, 0)),
            pl.BlockSpec((1, D), lambda i: (0, 0)),
            pl.BlockSpec((D, D), lambda i: (0, 0)),
        ],
        out_specs=pl.BlockSpec((_R, D), lambda i: (i, 0)),
        out_shape=jax.ShapeDtypeStruct((N, D), jnp.float32),
    )(p0, p1, dv, b1, W2)


def _tc_post_body(q0_ref, q1_ref, dv_ref, b_ref, o_ref):
    dinv = dv_ref[:, 0:1]
    o_ref[...] = (q0_ref[...] + q1_ref[...]) * dinv + b_ref[...]


def _tc_post(q0, q1, dv, b2):
    return pl.pallas_call(
        _tc_post_body,
        grid=(N // _R,),
        in_specs=[
            pl.BlockSpec((_R, D), lambda i: (i, 0)),
            pl.BlockSpec((_R, D), lambda i: (i, 0)),
            pl.BlockSpec((_R, 16), lambda i: (i, 0)),
            pl.BlockSpec((1, D), lambda i: (0, 0)),
        ],
        out_specs=pl.BlockSpec((_R, D), lambda i: (i, 0)),
        out_shape=jax.ShapeDtypeStruct((N, D), jnp.float32),
    )(q0, q1, dv, b2)


def kernel(x, edge_index, W1, b1, W2, b2):
    src2d = edge_index[0].reshape(TOTCH, C)
    dst2d = edge_index[1].reshape(TOTCH, C)
    ones128 = jnp.ones((C, D), jnp.float32)
    z128 = jnp.zeros((N, D), jnp.float32)
    b1r = b1.reshape(1, D)
    b2r = b2.reshape(1, D)

    h0, h1 = _sc_hist(dst2d, ones128, z128)
    g1, dv = _tc_pre(h0, h1, x, W1)
    p0, p1 = _sc_agg(src2d, dst2d, g1, z128)
    g2 = _tc_mid(p0, p1, dv, b1r, W2)
    q0, q1 = _sc_agg(src2d, dst2d, g2, z128)
    out = _tc_post(q0, q1, dv, b2r)
    return out


# speedup vs baseline: 25.9276x; 1.2902x over previous
import functools

import jax
import jax.numpy as jnp
from jax import lax
from jax.experimental import pallas as pl
from jax.experimental.pallas import tpu as pltpu
from jax.experimental.pallas import tpu_sc as plsc

N = 10000
E = 320000
D = 128

NC = 2
NS = 16
NW = NC * NS
EW = E // NW
C = 80
NCH = EW // C
RA = 624
TAIL = N - NS * RA

_MESH = plsc.VectorSubcoreMesh(
    core_axis_name="c", subcore_axis_name="s", num_cores=NC, num_subcores=NS)


def _sc_hist_body(dst_hbm, ones_hbm, z_hbm, out0, out1,
                  idx_d, ones_v, acc):
    cid = lax.axis_index("c")
    sid = lax.axis_index("s")
    wid = sid * NC + cid
    rsl = pl.ds(sid * RA, RA)
    tsl = pl.ds(NS * RA, TAIL)
    pltpu.sync_copy(z_hbm.at[rsl], acc.at[rsl])

    @pl.when(sid == NS - 1)
    def _():
        pltpu.sync_copy(z_hbm.at[tsl], acc.at[tsl])

    pltpu.sync_copy(dst_hbm.at[wid], idx_d)
    pltpu.sync_copy(ones_hbm, ones_v)
    plsc.subcore_barrier()

    def body(j, carry):
        pltpu.sync_copy(ones_v, acc.at[idx_d.at[j]], add=True)
        return carry

    lax.fori_loop(0, NCH, body, 0)
    plsc.subcore_barrier()

    @pl.when(cid == 0)
    def _():
        pltpu.sync_copy(acc.at[rsl], out0.at[rsl])

        @pl.when(sid == NS - 1)
        def _():
            pltpu.sync_copy(acc.at[tsl], out0.at[tsl])

    @pl.when(cid == 1)
    def _():
        pltpu.sync_copy(acc.at[rsl], out1.at[rsl])

        @pl.when(sid == NS - 1)
        def _():
            pltpu.sync_copy(acc.at[tsl], out1.at[tsl])


_sc_hist = pl.kernel(
    _sc_hist_body,
    out_type=[jax.ShapeDtypeStruct((N, D), jnp.float32),
              jax.ShapeDtypeStruct((N, D), jnp.float32)],
    mesh=_MESH,
    scratch_types=[
        pltpu.VMEM((NCH, C), jnp.int32),
        pltpu.VMEM((C, D), jnp.float32),
        pltpu.VMEM_SHARED((N, D), jnp.float32),
    ],
)


def _sc_agg_body(src_hbm, dst_hbm, g_hbm, z_hbm, out0, out1,
                 idx_s, idx_d, rows0, rows1, sem0, sem1, acc):
    cid = lax.axis_index("c")
    sid = lax.axis_index("s")
    wid = sid * NC + cid
    rsl = pl.ds(sid * RA, RA)
    tsl = pl.ds(NS * RA, TAIL)
    @pl.when(cid == 0)
    def _():
        pltpu.sync_copy(g_hbm.at[rsl], acc.at[rsl])

        @pl.when(sid == NS - 1)
        def _():
            pltpu.sync_copy(g_hbm.at[tsl], acc.at[tsl])

    @pl.when(cid == 1)
    def _():
        pltpu.sync_copy(z_hbm.at[rsl], acc.at[rsl])

        @pl.when(sid == NS - 1)
        def _():
            pltpu.sync_copy(z_hbm.at[tsl], acc.at[tsl])

    pltpu.sync_copy(src_hbm.at[pl.ds(wid * EW, EW)], idx_s)
    pltpu.sync_copy(dst_hbm.at[wid], idx_d)
    plsc.subcore_barrier()

    def _sidx(j):
        return idx_s.at[pl.ds(j * C, C)]

    pltpu.sync_copy(g_hbm.at[_sidx(0)], rows0)
    pltpu.sync_copy(rows0, acc.at[idx_d.at[0]], add=True)
    pltpu.async_copy(g_hbm.at[_sidx(1)], rows0, sem0)

    def body(t, carry):
        j0 = 2 * t + 1
        j1 = j0 + 1
        pltpu.async_copy(g_hbm.at[_sidx(j1)], rows1, sem1)
        pltpu.make_async_copy(g_hbm.at[_sidx(j0)], rows0, sem0).wait()
        pltpu.sync_copy(rows0, acc.at[idx_d.at[j0]], add=True)

        @pl.when(j1 + 1 < NCH)
        def _():
            pltpu.async_copy(g_hbm.at[_sidx(j1 + 1)], rows0, sem0)

        pltpu.make_async_copy(g_hbm.at[_sidx(j1)], rows1, sem1).wait()
        pltpu.sync_copy(rows1, acc.at[idx_d.at[j1]], add=True)
        return carry

    lax.fori_loop(0, (NCH - 1) // 2, body, 0)
    plsc.subcore_barrier()

    @pl.when(cid == 0)
    def _():
        pltpu.sync_copy(acc.at[rsl], out0.at[rsl])

        @pl.when(sid == NS - 1)
        def _():
            pltpu.sync_copy(acc.at[tsl], out0.at[tsl])

    @pl.when(cid == 1)
    def _():
        pltpu.sync_copy(acc.at[rsl], out1.at[rsl])

        @pl.when(sid == NS - 1)
        def _():
            pltpu.sync_copy(acc.at[tsl], out1.at[tsl])


_sc_agg = pl.kernel(
    _sc_agg_body,
    out_type=[jax.ShapeDtypeStruct((N, D), jnp.float32),
              jax.ShapeDtypeStruct((N, D), jnp.float32)],
    mesh=_MESH,
    scratch_types=[
        pltpu.VMEM((EW,), jnp.int32),
        pltpu.VMEM((NCH, C), jnp.int32),
        pltpu.VMEM((C, D), jnp.float32),
        pltpu.VMEM((C, D), jnp.float32),
        pltpu.SemaphoreType.DMA,
        pltpu.SemaphoreType.DMA,
        pltpu.VMEM_SHARED((N, D), jnp.float32),
    ],
)


_R = 1000


def _tc_pre_body(h0_ref, h1_ref, x_ref, w_ref, g_ref, dv_ref):
    deg = h0_ref[:, 0:1] + h1_ref[:, 0:1] + 1.0
    dinv = lax.rsqrt(jnp.maximum(deg, 1.0))
    h = jnp.dot(x_ref[...], w_ref[...], preferred_element_type=jnp.float32)
    g_ref[...] = h * dinv
    dv_ref[...] = jnp.broadcast_to(dinv, (_R, 16))


def _tc_pre(h0, h1, x, W1):
    return pl.pallas_call(
        _tc_pre_body,
        grid=(N // _R,),
        in_specs=[
            pl.BlockSpec((_R, D), lambda i: (i, 0)),
            pl.BlockSpec((_R, D), lambda i: (i, 0)),
            pl.BlockSpec((_R, D), lambda i: (i, 0)),
            pl.BlockSpec((D, D), lambda i: (0, 0)),
        ],
        out_specs=[pl.BlockSpec((_R, D), lambda i: (i, 0)),
                   pl.BlockSpec((_R, 16), lambda i: (i, 0))],
        out_shape=[jax.ShapeDtypeStruct((N, D), jnp.float32),
                   jax.ShapeDtypeStruct((N, 16), jnp.float32)],
    )(h0, h1, x, W1)


def _tc_mid_body(p0_ref, p1_ref, dv_ref, b_ref, w_ref, g_ref):
    dinv = dv_ref[:, 0:1]
    z = jnp.maximum((p0_ref[...] + p1_ref[...]) * dinv + b_ref[...], 0.0)
    g_ref[...] = jnp.dot(
        z, w_ref[...], preferred_element_type=jnp.float32) * dinv


def _tc_mid(p0, p1, dv, b1, W2):
    return pl.pallas_call(
        _tc_mid_body,
        grid=(N // _R,),
        in_specs=[
            pl.BlockSpec((_R, D), lambda i: (i, 0)),
            pl.BlockSpec((_R, D), lambda i: (i, 0)),
            pl.BlockSpec((_R, 16), lambda i: (i, 0)),
            pl.BlockSpec((1, D), lambda i: (0, 0)),
            pl.BlockSpec((D, D), lambda i: (0, 0)),
        ],
        out_specs=pl.BlockSpec((_R, D), lambda i: (i, 0)),
        out_shape=jax.ShapeDtypeStruct((N, D), jnp.float32),
    )(p0, p1, dv, b1, W2)


def _tc_post_body(q0_ref, q1_ref, dv_ref, b_ref, o_ref):
    dinv = dv_ref[:, 0:1]
    o_ref[...] = (q0_ref[...] + q1_ref[...]) * dinv + b_ref[...]


def _tc_post(q0, q1, dv, b2):
    return pl.pallas_call(
        _tc_post_body,
        grid=(N // _R,),
        in_specs=[
            pl.BlockSpec((_R, D), lambda i: (i, 0)),
            pl.BlockSpec((_R, D), lambda i: (i, 0)),
            pl.BlockSpec((_R, 16), lambda i: (i, 0)),
            pl.BlockSpec((1, D), lambda i: (0, 0)),
        ],
        out_specs=pl.BlockSpec((_R, D), lambda i: (i, 0)),
        out_shape=jax.ShapeDtypeStruct((N, D), jnp.float32),
    )(q0, q1, dv, b2)


def kernel(x, edge_index, W1, b1, W2, b2):
    src1d = edge_index[0]
    dst3d = edge_index[1].reshape(NW, NCH, C)
    ones128 = jnp.ones((C, D), jnp.float32)
    z128 = jnp.zeros((N, D), jnp.float32)
    b1r = b1.reshape(1, D)
    b2r = b2.reshape(1, D)

    h0, h1 = _sc_hist(dst3d, ones128, z128)
    g1, dv = _tc_pre(h0, h1, x, W1)
    p0, p1 = _sc_agg(src1d, dst3d, g1, z128)
    g2 = _tc_mid(p0, p1, dv, b1r, W2)
    q0, q1 = _sc_agg(src1d, dst3d, g2, z128)
    out = _tc_post(q0, q1, dv, b2r)
    return out


# speedup vs baseline: 30.1417x; 1.1625x over previous
import functools

import jax
import jax.numpy as jnp
from jax import lax
from jax.experimental import pallas as pl
from jax.experimental.pallas import tpu as pltpu
from jax.experimental.pallas import tpu_sc as plsc

N = 10000
E = 320000
D = 128

NC = 2
NS = 16
NW = NC * NS
EW = E // NW
C = 80
NCH = EW // C
RA = 624
TAIL = N - NS * RA

_MESH = plsc.VectorSubcoreMesh(
    core_axis_name="c", subcore_axis_name="s", num_cores=NC, num_subcores=NS)


HR = N // 16


def _sc_hist_body(dst_hbm, zh_hbm, out, idx_d, hist2):
    cid = lax.axis_index("c")
    sid = lax.axis_index("s")
    wid = sid * NC + cid
    pltpu.sync_copy(zh_hbm, hist2)
    pltpu.sync_copy(dst_hbm.at[pl.ds(wid * EW, EW)], idx_d)
    iota = lax.iota(jnp.int32, 16)
    lane_lo = jnp.where(iota < 8, iota, 0)
    lane_hi = jnp.where(iota >= 8, iota - 8, 0)
    mask_lo = iota < 8
    mask_hi = iota >= 8
    ones_v = jnp.ones((16,), jnp.float32)

    def body(i, carry):
        idx = idx_d[pl.ds(i * 16, 16)]
        row = lax.shift_right_logical(idx, 4)
        colbase = lax.shift_left(jnp.bitwise_and(idx, 15), 3)
        plsc.addupdate_scatter(
            hist2, [row, colbase + lane_lo], ones_v, mask=mask_lo)
        plsc.addupdate_scatter(
            hist2, [row, colbase + lane_hi], ones_v, mask=mask_hi)
        return carry

    lax.fori_loop(0, EW // 16, body, 0)
    pltpu.sync_copy(hist2, out.at[wid])


_sc_hist = pl.kernel(
    _sc_hist_body,
    out_type=jax.ShapeDtypeStruct((NW, HR, 128), jnp.float32),
    mesh=_MESH,
    scratch_types=[
        pltpu.VMEM((EW,), jnp.int32),
        pltpu.VMEM((HR, 128), jnp.float32),
    ],
    compiler_params=pltpu.CompilerParams(needs_layout_passes=False),
)


def _sc_agg_body(src_hbm, dst_hbm, g_hbm, z_hbm, out0, out1,
                 idx_s, idx_d, rows0, rows1, sem0, sem1, acc):
    cid = lax.axis_index("c")
    sid = lax.axis_index("s")
    wid = sid * NC + cid
    rsl = pl.ds(sid * RA, RA)
    tsl = pl.ds(NS * RA, TAIL)
    @pl.when(cid == 0)
    def _():
        pltpu.sync_copy(g_hbm.at[rsl], acc.at[rsl])

        @pl.when(sid == NS - 1)
        def _():
            pltpu.sync_copy(g_hbm.at[tsl], acc.at[tsl])

    @pl.when(cid == 1)
    def _():
        pltpu.sync_copy(z_hbm.at[rsl], acc.at[rsl])

        @pl.when(sid == NS - 1)
        def _():
            pltpu.sync_copy(z_hbm.at[tsl], acc.at[tsl])

    pltpu.sync_copy(src_hbm.at[pl.ds(wid * EW, EW)], idx_s)
    pltpu.sync_copy(dst_hbm.at[wid], idx_d)
    plsc.subcore_barrier()

    def _sidx(j):
        return idx_s.at[pl.ds(j * C, C)]

    pltpu.sync_copy(g_hbm.at[_sidx(0)], rows0)
    pltpu.sync_copy(rows0, acc.at[idx_d.at[0]], add=True)
    pltpu.async_copy(g_hbm.at[_sidx(1)], rows0, sem0)

    def body(t, carry):
        j0 = 2 * t + 1
        j1 = j0 + 1
        pltpu.async_copy(g_hbm.at[_sidx(j1)], rows1, sem1)
        pltpu.make_async_copy(g_hbm.at[_sidx(j0)], rows0, sem0).wait()
        pltpu.sync_copy(rows0, acc.at[idx_d.at[j0]], add=True)

        @pl.when(j1 + 1 < NCH)
        def _():
            pltpu.async_copy(g_hbm.at[_sidx(j1 + 1)], rows0, sem0)

        pltpu.make_async_copy(g_hbm.at[_sidx(j1)], rows1, sem1).wait()
        pltpu.sync_copy(rows1, acc.at[idx_d.at[j1]], add=True)
        return carry

    lax.fori_loop(0, (NCH - 1) // 2, body, 0)
    plsc.subcore_barrier()

    @pl.when(cid == 0)
    def _():
        pltpu.sync_copy(acc.at[rsl], out0.at[rsl])

        @pl.when(sid == NS - 1)
        def _():
            pltpu.sync_copy(acc.at[tsl], out0.at[tsl])

    @pl.when(cid == 1)
    def _():
        pltpu.sync_copy(acc.at[rsl], out1.at[rsl])

        @pl.when(sid == NS - 1)
        def _():
            pltpu.sync_copy(acc.at[tsl], out1.at[tsl])


_sc_agg = pl.kernel(
    _sc_agg_body,
    out_type=[jax.ShapeDtypeStruct((N, D), jnp.float32),
              jax.ShapeDtypeStruct((N, D), jnp.float32)],
    mesh=_MESH,
    scratch_types=[
        pltpu.VMEM((EW,), jnp.int32),
        pltpu.VMEM((NCH, C), jnp.int32),
        pltpu.VMEM((C, D), jnp.float32),
        pltpu.VMEM((C, D), jnp.float32),
        pltpu.SemaphoreType.DMA,
        pltpu.SemaphoreType.DMA,
        pltpu.VMEM_SHARED((N, D), jnp.float32),
    ],
)


_R = 1000


def _tc_pre_body(hist_ref, x_ref, w_ref, g_ref, dv_ref):
    hs = jnp.sum(hist_ref[...], axis=0)
    deg = jnp.sum(hs.reshape(HR, 16, 8), axis=2)
    deg = deg.reshape(N, 1) + 1.0
    dinv = lax.rsqrt(jnp.maximum(deg, 1.0))
    h = jnp.dot(x_ref[...], w_ref[...], preferred_element_type=jnp.float32)
    g_ref[...] = h * dinv
    dv_ref[...] = jnp.broadcast_to(dinv, (N, 16))


def _tc_pre(hist, x, W1):
    return pl.pallas_call(
        _tc_pre_body,
        out_shape=[jax.ShapeDtypeStruct((N, D), jnp.float32),
                   jax.ShapeDtypeStruct((N, 16), jnp.float32)],
    )(hist, x, W1)


def _tc_mid_body(p0_ref, p1_ref, dv_ref, b_ref, w_ref, g_ref):
    dinv = dv_ref[:, 0:1]
    z = jnp.maximum((p0_ref[...] + p1_ref[...]) * dinv + b_ref[...], 0.0)
    g_ref[...] = jnp.dot(
        z, w_ref[...], preferred_element_type=jnp.float32) * dinv


def _tc_mid(p0, p1, dv, b1, W2):
    return pl.pallas_call(
        _tc_mid_body,
        grid=(N // _R,),
        in_specs=[
            pl.BlockSpec((_R, D), lambda i: (i, 0)),
            pl.BlockSpec((_R, D), lambda i: (i, 0)),
            pl.BlockSpec((_R, 16), lambda i: (i, 0)),
            pl.BlockSpec((1, D), lambda i: (0, 0)),
            pl.BlockSpec((D, D), lambda i: (0, 0)),
        ],
        out_specs=pl.BlockSpec((_R, D), lambda i: (i, 0)),
        out_shape=jax.ShapeDtypeStruct((N, D), jnp.float32),
    )(p0, p1, dv, b1, W2)


def _tc_post_body(q0_ref, q1_ref, dv_ref, b_ref, o_ref):
    dinv = dv_ref[:, 0:1]
    o_ref[...] = (q0_ref[...] + q1_ref[...]) * dinv + b_ref[...]


def _tc_post(q0, q1, dv, b2):
    return pl.pallas_call(
        _tc_post_body,
        grid=(N // _R,),
        in_specs=[
            pl.BlockSpec((_R, D), lambda i: (i, 0)),
            pl.BlockSpec((_R, D), lambda i: (i, 0)),
            pl.BlockSpec((_R, 16), lambda i: (i, 0)),
            pl.BlockSpec((1, D), lambda i: (0, 0)),
        ],
        out_specs=pl.BlockSpec((_R, D), lambda i: (i, 0)),
        out_shape=jax.ShapeDtypeStruct((N, D), jnp.float32),
    )(q0, q1, dv, b2)


def kernel(x, edge_index, W1, b1, W2, b2):
    src1d = edge_index[0]
    dst1d = edge_index[1]
    dst3d = edge_index[1].reshape(NW, NCH, C)
    zh = jnp.zeros((HR, 128), jnp.float32)
    z128 = jnp.zeros((N, D), jnp.float32)
    b1r = b1.reshape(1, D)
    b2r = b2.reshape(1, D)

    hist = _sc_hist(dst1d, zh)
    g1, dv = _tc_pre(hist, x, W1)
    p0, p1 = _sc_agg(src1d, dst3d, g1, z128)
    g2 = _tc_mid(p0, p1, dv, b1r, W2)
    q0, q1 = _sc_agg(src1d, dst3d, g2, z128)
    out = _tc_post(q0, q1, dv, b2r)
    return out
